# Initial kernel scaffold; baseline (speedup 1.0000x reference)
import jax
import jax.numpy as jnp
from jax.experimental import pallas as pl


def kernel(species, positions, edges, potential, efield, Wspecies, Wv, We, W1, b1, W2, Wsc, Wlin, Wang):
    raise NotImplementedError("write your pallas kernel here")


# speedup vs baseline: 2.4903x; 2.4903x over previous
import functools
import math

import jax
import jax.numpy as jnp
import numpy as np
from jax import lax
from jax.experimental import pallas as pl
from jax.experimental.pallas import tpu as pltpu
from jax.experimental.pallas import tpu_sc as plsc

_NC = 2
_NS = 16
_L = 16
_NW = _NC * _NS

_F = 128
_NUM_BASIS = 20
_HID = 64
_CUTOFF = 4.0
_INV_SQRT_NN = 1.0 / math.sqrt(32.0)

_f32 = jnp.float32


def _r2_sc(pos_flat, src, dst):
    n = pos_flat.shape[0] // 4
    e = src.shape[0]
    epw = e // _NW
    c_sz = 400
    nchunk = epw // c_sz
    ngrp = c_sz // _L

    mesh = plsc.VectorSubcoreMesh(
        core_axis_name="c", subcore_axis_name="s",
        num_cores=_NC, num_subcores=_NS)

    @functools.partial(
        pl.kernel, mesh=mesh,
        compiler_params=pltpu.CompilerParams(needs_layout_passes=False),
        out_type=jax.ShapeDtypeStruct((e,), _f32),
        scratch_types=[
            pltpu.VMEM((n * 4,), _f32),
            pltpu.VMEM((c_sz,), jnp.int32),
            pltpu.VMEM((c_sz,), jnp.int32),
            pltpu.VMEM((c_sz,), _f32),
        ],
    )
    def k(pos_hbm, src_hbm, dst_hbm, out_hbm, pos_v, si_v, di_v, o_v):
        wid = lax.axis_index("s") * _NC + lax.axis_index("c")
        pltpu.sync_copy(pos_hbm, pos_v)
        base_t = wid * epw

        def chunk(j, carry):
            base = base_t + j * c_sz
            pltpu.sync_copy(src_hbm.at[pl.ds(base, c_sz)], si_v)
            pltpu.sync_copy(dst_hbm.at[pl.ds(base, c_sz)], di_v)

            def grp(g, carry2):
                s16 = si_v[pl.ds(g * _L, _L)] * 4
                d16 = di_v[pl.ds(g * _L, _L)] * 4
                acc = jnp.zeros((_L,), _f32)
                for comp in range(3):
                    pa = plsc.load_gather(pos_v, [d16 + comp])
                    pb = plsc.load_gather(pos_v, [s16 + comp])
                    dd = pa - pb
                    acc = acc + dd * dd
                o_v[pl.ds(g * _L, _L)] = acc
                return carry2

            lax.fori_loop(0, ngrp, grp, 0)
            pltpu.sync_copy(o_v, out_hbm.at[pl.ds(base, c_sz)])
            return carry

        lax.fori_loop(0, nchunk, chunk, 0)

    return k(pos_flat, src, dst)


def _w_tc(r2, W1, b1, W2, vals, ascale, cshift, inv_step):
    e = r2.shape[0]
    layers = W1.shape[0]
    eb = 1280
    nbe = e // eb
    r2_3 = r2.reshape(nbe, 1, eb)
    b1_3 = b1.reshape(layers, 1, _HID)

    def body(r2_ref, vals_ref, a_ref, c_ref, w1_ref, b1_ref, w2_ref, o_ref):
        r2b = r2_ref[0, 0, :]
        r = jnp.sqrt(r2b + 1e-12)
        d = (r[:, None] - vals_ref[0, :][None, :]) * inv_step
        b = jnp.exp(-d * d)
        rb = b * a_ref[0, :][None, :] - c_ref[0, :][None, :]
        h = jnp.dot(rb, w1_ref[0], preferred_element_type=_f32) \
            + b1_ref[0, 0, :][None, :]
        h = h * jax.nn.sigmoid(h)
        o_ref[0] = jnp.dot(h, w2_ref[0], preferred_element_type=_f32)

    return pl.pallas_call(
        body,
        grid=(layers, nbe),
        in_specs=[
            pl.BlockSpec((1, 1, eb), lambda i, j: (j, 0, 0)),
            pl.BlockSpec((1, _NUM_BASIS), lambda i, j: (0, 0)),
            pl.BlockSpec((1, _NUM_BASIS), lambda i, j: (0, 0)),
            pl.BlockSpec((1, _NUM_BASIS), lambda i, j: (0, 0)),
            pl.BlockSpec((1, _NUM_BASIS, _HID), lambda i, j: (i, 0, 0)),
            pl.BlockSpec((1, 1, _HID), lambda i, j: (i, 0, 0)),
            pl.BlockSpec((1, _HID, _F), lambda i, j: (i, 0, 0)),
        ],
        out_specs=pl.BlockSpec((1, eb, _F), lambda i, j: (i, j, 0)),
        out_shape=jax.ShapeDtypeStruct((layers, e, _F), _f32),
    )(r2_3, vals, ascale, cshift, W1, b1_3, W2)


def _emb_tc(species, potential, efield, Wspecies, Wv, We):
    n = species.shape[0]
    blk = 400
    nb = n // blk
    sp3 = species.reshape(nb, 1, blk)
    pot3 = potential.reshape(nb, 1, blk)
    efx = efield[:, 0].reshape(nb, 1, blk)
    efy = efield[:, 1].reshape(nb, 1, blk)
    efz = efield[:, 2].reshape(nb, 1, blk)
    wsp = jnp.pad(Wspecies, ((0, _F - Wspecies.shape[0]), (0, 0)))

    def body(sp_ref, pot_ref, ex_ref, ey_ref, ez_ref, wsp_ref, wv_ref,
             we_ref, o_ref):
        sp = sp_ref[0, 0, :]
        oh = (sp[:, None] == lax.broadcasted_iota(jnp.int32, (blk, _F), 1))
        x = jnp.dot(oh.astype(_f32), wsp_ref[...],
                    preferred_element_type=_f32)
        x = x + pot_ref[0, 0, :][:, None] * wv_ref[0, :][None, :]
        x = x + ex_ref[0, 0, :][:, None] * we_ref[0, :][None, :]
        x = x + ey_ref[0, 0, :][:, None] * we_ref[1, :][None, :]
        x = x + ez_ref[0, 0, :][:, None] * we_ref[2, :][None, :]
        o_ref[...] = x

    return pl.pallas_call(
        body,
        grid=(nb,),
        in_specs=[
            pl.BlockSpec((1, 1, blk), lambda j: (j, 0, 0)),
            pl.BlockSpec((1, 1, blk), lambda j: (j, 0, 0)),
            pl.BlockSpec((1, 1, blk), lambda j: (j, 0, 0)),
            pl.BlockSpec((1, 1, blk), lambda j: (j, 0, 0)),
            pl.BlockSpec((1, 1, blk), lambda j: (j, 0, 0)),
            pl.BlockSpec((_F, _F), lambda j: (0, 0)),
            pl.BlockSpec((1, _F), lambda j: (0, 0)),
            pl.BlockSpec((3, _F), lambda j: (0, 0)),
        ],
        out_specs=pl.BlockSpec((blk, _F), lambda j: (j, 0)),
        out_shape=jax.ShapeDtypeStruct((n, _F), _f32),
    )(sp3, pot3, efx, efy, efz, wsp, Wv, We)


def _gms_sc(x, w, src, dst, zeros_blk, c_sz):
    n = x.shape[0]
    e = w.shape[0]
    epw = e // _NW
    npc = epw // c_sz
    rows_pt = (n // _NS) // 8 * 8
    rows_rem = n - _NS * rows_pt

    mesh = plsc.VectorSubcoreMesh(
        core_axis_name="c", subcore_axis_name="s",
        num_cores=_NC, num_subcores=_NS)

    @functools.partial(
        pl.kernel, mesh=mesh,
        compiler_params=pltpu.CompilerParams(needs_layout_passes=False),
        out_type=jax.ShapeDtypeStruct((_NC, n, _F), _f32),
        scratch_types=[
            pltpu.VMEM((c_sz,), jnp.int32),
            pltpu.VMEM((c_sz,), jnp.int32),
            pltpu.VMEM((c_sz, _F), _f32),
            pltpu.VMEM((c_sz, _F), _f32),
            pltpu.VMEM_SHARED((n, _F), _f32),
            pltpu.SemaphoreType.DMA,
        ],
    )
    def k(x_hbm, w_hbm, src_hbm, dst_hbm, z_hbm, out_hbm,
          si_v, di_v, xg_v, wv_v, agg_sh, gsem):
        cid = lax.axis_index("c")
        sid = lax.axis_index("s")
        wid = sid * _NC + cid
        pltpu.sync_copy(z_hbm.at[pl.ds(0, rows_pt)],
                        agg_sh.at[pl.ds(sid * rows_pt, rows_pt)])

        @pl.when(sid == 0)
        def _zero_tail():
            pltpu.sync_copy(z_hbm.at[pl.ds(0, rows_rem)],
                            agg_sh.at[pl.ds(_NS * rows_pt, rows_rem)])

        plsc.subcore_barrier()

        def chunk(j, carry):
            base = wid * epw + j * c_sz
            pltpu.sync_copy(src_hbm.at[pl.ds(base, c_sz)], si_v)
            pltpu.sync_copy(dst_hbm.at[pl.ds(base, c_sz)], di_v)
            pltpu.async_copy(x_hbm.at[si_v], xg_v, gsem).wait()
            pltpu.sync_copy(w_hbm.at[pl.ds(base, c_sz)], wv_v)

            def mrow(r, carry2):
                for kk in range(_F // _L):
                    sl = pl.ds(kk * _L, _L)
                    xg_v[r, sl] = xg_v[r, sl] * wv_v[r, sl]
                return carry2

            lax.fori_loop(0, c_sz, mrow, 0)
            pltpu.sync_copy(xg_v, agg_sh.at[di_v], add=True)
            return carry

        lax.fori_loop(0, npc, chunk, 0)
        plsc.subcore_barrier()
        pltpu.sync_copy(
            agg_sh.at[pl.ds(sid * rows_pt, rows_pt)],
            out_hbm.at[cid, pl.ds(sid * rows_pt, rows_pt)])

        @pl.when(sid == 0)
        def _write_tail():
            pltpu.sync_copy(
                agg_sh.at[pl.ds(_NS * rows_pt, rows_rem)],
                out_hbm.at[cid, pl.ds(_NS * rows_pt, rows_rem)])

    return k(x, w, src, dst, zeros_blk)


def _comb_tc(x, agg2, Wsc_i, Wlin_i, wang_row):
    n = x.shape[0]
    blk = 400
    nb = n // blk

    def body(x_ref, agg_ref, wsc_ref, wlin_ref, wang_ref, o_ref):
        xv = x_ref[...]
        agg = (agg_ref[0] + agg_ref[1]) * _INV_SQRT_NN
        sc = jnp.dot(xv, wsc_ref[...], preferred_element_type=_f32)
        conv = jnp.dot(agg, wlin_ref[...], preferred_element_type=_f32)
        ang = 0.1 * jnp.sum(agg * wang_ref[...], axis=1, keepdims=True)
        o_ref[...] = jnp.cos(ang) * sc + jnp.sin(ang) * conv

    return pl.pallas_call(
        body,
        grid=(nb,),
        in_specs=[
            pl.BlockSpec((blk, _F), lambda j: (j, 0)),
            pl.BlockSpec((_NC, blk, _F), lambda j: (0, j, 0)),
            pl.BlockSpec((_F, _F), lambda j: (0, 0)),
            pl.BlockSpec((_F, _F), lambda j: (0, 0)),
            pl.BlockSpec((1, _F), lambda j: (0, 0)),
        ],
        out_specs=pl.BlockSpec((blk, _F), lambda j: (j, 0)),
        out_shape=jax.ShapeDtypeStruct((n, _F), _f32),
    )(x, agg2, Wsc_i, Wlin_i, wang_row)


def kernel(species, positions, edges, potential, efield,
           Wspecies, Wv, We, W1, b1, W2, Wsc, Wlin, Wang):
    n = species.shape[0]
    e = edges.shape[0]
    layers = W1.shape[0]

    src = edges[:, 0]
    dst = edges[:, 1]
    pos_flat = jnp.pad(positions, ((0, 0), (0, 1))).reshape(-1)

    r2 = _r2_sc(pos_flat, src, dst)

    vals_np = np.linspace(0.0, _CUTOFF, _NUM_BASIS, dtype=np.float32)
    step = float(vals_np[1] - vals_np[0])
    inv_step = 1.0 / step
    vals = jnp.asarray(vals_np).reshape(1, _NUM_BASIS)
    rs = jnp.linspace(0.0, _CUTOFF, 4001)[1:]
    dsamp = (rs[:, None] - vals[0][None, :]) * inv_step
    bs = jnp.exp(-dsamp * dsamp) / 1.12
    mean = jnp.mean(bs, axis=0)
    std = jnp.std(bs, axis=0, ddof=1)
    ascale = ((1.0 / 1.12) / std).reshape(1, _NUM_BASIS)
    cshift = (mean / std).reshape(1, _NUM_BASIS)

    w_all = _w_tc(r2, W1, b1, W2, vals, ascale, cshift, inv_step)
    x = _emb_tc(species, potential, efield, Wspecies, Wv, We)

    zeros_blk = jnp.zeros((n // _NS, _F), _f32)

    for i in range(layers):
        agg2 = _gms_sc(x, w_all[i], src, dst, zeros_blk, 80)
        x = _comb_tc(x, agg2, Wsc[i], Wlin[i], Wang[i].reshape(1, _F))
    return x


# speedup vs baseline: 4.0159x; 1.6126x over previous
import functools
import math

import jax
import jax.numpy as jnp
import numpy as np
from jax import lax
from jax.experimental import pallas as pl
from jax.experimental.pallas import tpu as pltpu
from jax.experimental.pallas import tpu_sc as plsc

_NC = 2
_NS = 16
_L = 16
_NW = _NC * _NS

_F = 128
_NUM_BASIS = 20
_HID = 64
_CUTOFF = 4.0
_INV_SQRT_NN = 1.0 / math.sqrt(32.0)

_f32 = jnp.float32


def _r2_sc(pos_flat, src, dst):
    n = pos_flat.shape[0] // 4
    e = src.shape[0]
    epw = e // _NW
    c_sz = 400
    nchunk = epw // c_sz
    ngrp = c_sz // _L

    mesh = plsc.VectorSubcoreMesh(
        core_axis_name="c", subcore_axis_name="s",
        num_cores=_NC, num_subcores=_NS)

    @functools.partial(
        pl.kernel, mesh=mesh,
        compiler_params=pltpu.CompilerParams(needs_layout_passes=False),
        out_type=jax.ShapeDtypeStruct((e,), _f32),
        scratch_types=[
            pltpu.VMEM((n * 4,), _f32),
            pltpu.VMEM((c_sz,), jnp.int32),
            pltpu.VMEM((c_sz,), jnp.int32),
            pltpu.VMEM((c_sz,), _f32),
        ],
    )
    def k(pos_hbm, src_hbm, dst_hbm, out_hbm, pos_v, si_v, di_v, o_v):
        wid = lax.axis_index("s") * _NC + lax.axis_index("c")
        pltpu.sync_copy(pos_hbm, pos_v)
        base_t = wid * epw

        def chunk(j, carry):
            base = base_t + j * c_sz
            pltpu.sync_copy(src_hbm.at[pl.ds(base, c_sz)], si_v)
            pltpu.sync_copy(dst_hbm.at[pl.ds(base, c_sz)], di_v)

            def grp(g, carry2):
                s16 = si_v[pl.ds(g * _L, _L)] * 4
                d16 = di_v[pl.ds(g * _L, _L)] * 4
                acc = jnp.zeros((_L,), _f32)
                for comp in range(3):
                    pa = plsc.load_gather(pos_v, [d16 + comp])
                    pb = plsc.load_gather(pos_v, [s16 + comp])
                    dd = pa - pb
                    acc = acc + dd * dd
                o_v[pl.ds(g * _L, _L)] = acc
                return carry2

            lax.fori_loop(0, ngrp, grp, 0)
            pltpu.sync_copy(o_v, out_hbm.at[pl.ds(base, c_sz)])
            return carry

        lax.fori_loop(0, nchunk, chunk, 0)

    return k(pos_flat, src, dst)


def _w_tc(r2, W1, b1, W2, vals, ascale, cshift, inv_step):
    e = r2.shape[0]
    layers = W1.shape[0]
    eb = 1280
    nbe = e // eb
    r2_3 = r2.reshape(nbe, 1, eb)
    b1_3 = b1.reshape(layers, 1, _HID)

    def body(r2_ref, vals_ref, a_ref, c_ref, w1_ref, b1_ref, w2_ref, o_ref):
        r2b = r2_ref[0, 0, :]
        r = jnp.sqrt(r2b + 1e-12)
        d = (r[:, None] - vals_ref[0, :][None, :]) * inv_step
        b = jnp.exp(-d * d)
        rb = b * a_ref[0, :][None, :] - c_ref[0, :][None, :]
        h = jnp.dot(rb, w1_ref[0], preferred_element_type=_f32) \
            + b1_ref[0, 0, :][None, :]
        h = h * jax.nn.sigmoid(h)
        o_ref[0] = jnp.dot(h, w2_ref[0], preferred_element_type=_f32)

    return pl.pallas_call(
        body,
        grid=(layers, nbe),
        in_specs=[
            pl.BlockSpec((1, 1, eb), lambda i, j: (j, 0, 0)),
            pl.BlockSpec((1, _NUM_BASIS), lambda i, j: (0, 0)),
            pl.BlockSpec((1, _NUM_BASIS), lambda i, j: (0, 0)),
            pl.BlockSpec((1, _NUM_BASIS), lambda i, j: (0, 0)),
            pl.BlockSpec((1, _NUM_BASIS, _HID), lambda i, j: (i, 0, 0)),
            pl.BlockSpec((1, 1, _HID), lambda i, j: (i, 0, 0)),
            pl.BlockSpec((1, _HID, _F), lambda i, j: (i, 0, 0)),
        ],
        out_specs=pl.BlockSpec((1, eb, _F), lambda i, j: (i, j, 0)),
        out_shape=jax.ShapeDtypeStruct((layers, e, _F), _f32),
    )(r2_3, vals, ascale, cshift, W1, b1_3, W2)


def _emb_tc(species, potential, efield, Wspecies, Wv, We):
    n = species.shape[0]
    blk = 400
    nb = n // blk
    sp3 = species.reshape(nb, 1, blk)
    pot3 = potential.reshape(nb, 1, blk)
    efx = efield[:, 0].reshape(nb, 1, blk)
    efy = efield[:, 1].reshape(nb, 1, blk)
    efz = efield[:, 2].reshape(nb, 1, blk)
    wsp = jnp.pad(Wspecies, ((0, _F - Wspecies.shape[0]), (0, 0)))

    def body(sp_ref, pot_ref, ex_ref, ey_ref, ez_ref, wsp_ref, wv_ref,
             we_ref, o_ref):
        sp = sp_ref[0, 0, :]
        oh = (sp[:, None] == lax.broadcasted_iota(jnp.int32, (blk, _F), 1))
        x = jnp.dot(oh.astype(_f32), wsp_ref[...],
                    preferred_element_type=_f32)
        x = x + pot_ref[0, 0, :][:, None] * wv_ref[0, :][None, :]
        x = x + ex_ref[0, 0, :][:, None] * we_ref[0, :][None, :]
        x = x + ey_ref[0, 0, :][:, None] * we_ref[1, :][None, :]
        x = x + ez_ref[0, 0, :][:, None] * we_ref[2, :][None, :]
        o_ref[...] = x

    return pl.pallas_call(
        body,
        grid=(nb,),
        in_specs=[
            pl.BlockSpec((1, 1, blk), lambda j: (j, 0, 0)),
            pl.BlockSpec((1, 1, blk), lambda j: (j, 0, 0)),
            pl.BlockSpec((1, 1, blk), lambda j: (j, 0, 0)),
            pl.BlockSpec((1, 1, blk), lambda j: (j, 0, 0)),
            pl.BlockSpec((1, 1, blk), lambda j: (j, 0, 0)),
            pl.BlockSpec((_F, _F), lambda j: (0, 0)),
            pl.BlockSpec((1, _F), lambda j: (0, 0)),
            pl.BlockSpec((3, _F), lambda j: (0, 0)),
        ],
        out_specs=pl.BlockSpec((blk, _F), lambda j: (j, 0)),
        out_shape=jax.ShapeDtypeStruct((n, _F), _f32),
    )(sp3, pot3, efx, efy, efz, wsp, Wv, We)


def _gms_sc(x, w, src, dst, zeros_blk, c_sz):
    n = x.shape[0]
    e = w.shape[0]
    epw = e // _NW
    npc = epw // c_sz
    assert npc * c_sz == epw and npc % 4 == 1
    rows_pt = (n // _NS) // 8 * 8
    rows_rem = n - _NS * rows_pt

    mesh = plsc.VectorSubcoreMesh(
        core_axis_name="c", subcore_axis_name="s",
        num_cores=_NC, num_subcores=_NS)

    @functools.partial(
        pl.kernel, mesh=mesh,
        compiler_params=pltpu.CompilerParams(needs_layout_passes=False),
        out_type=jax.ShapeDtypeStruct((_NC, n, _F), _f32),
        scratch_types=[
            [pltpu.VMEM((c_sz,), jnp.int32) for _ in range(4)],
            [pltpu.VMEM((c_sz,), jnp.int32) for _ in range(4)],
            pltpu.VMEM((c_sz, _F), _f32),
            pltpu.VMEM((c_sz, _F), _f32),
            pltpu.VMEM((c_sz, _F), _f32),
            pltpu.VMEM((c_sz, _F), _f32),
            pltpu.VMEM_SHARED((n, _F), _f32),
            [pltpu.SemaphoreType.DMA for _ in range(4)],
            [pltpu.SemaphoreType.DMA for _ in range(4)],
            pltpu.SemaphoreType.DMA,
            pltpu.SemaphoreType.DMA,
            pltpu.SemaphoreType.DMA,
            pltpu.SemaphoreType.DMA,
        ],
    )
    def k(x_hbm, w_hbm, src_hbm, dst_hbm, z_hbm, out_hbm,
          si_v, di_v, xg_a, wv_a, xg_b, wv_b, agg_sh,
          ssi, sdi, sg_a, sw_a, sg_b, sw_b):
        cid = lax.axis_index("c")
        sid = lax.axis_index("s")
        wid = sid * _NC + cid
        base_t = wid * epw
        pltpu.sync_copy(z_hbm.at[pl.ds(0, rows_pt)],
                        agg_sh.at[pl.ds(sid * rows_pt, rows_pt)])

        @pl.when(sid == 0)
        def _zero_tail():
            pltpu.sync_copy(z_hbm.at[pl.ds(0, rows_rem)],
                            agg_sh.at[pl.ds(_NS * rows_pt, rows_rem)])

        plsc.subcore_barrier()

        def fire_idx(j, b):
            sl = pl.ds(base_t + j * c_sz, c_sz)
            pltpu.async_copy(src_hbm.at[sl], si_v[b], ssi[b])
            pltpu.async_copy(dst_hbm.at[sl], di_v[b], sdi[b])

        def wait_idx(j, b):
            sl = pl.ds(base_t + j * c_sz, c_sz)
            pltpu.make_async_copy(src_hbm.at[sl], si_v[b], ssi[b]).wait()
            pltpu.make_async_copy(dst_hbm.at[sl], di_v[b], sdi[b]).wait()

        def fire_gw(j, b, xg, wv, sg, sw):
            pltpu.async_copy(x_hbm.at[si_v[b]], xg, sg)
            pltpu.async_copy(w_hbm.at[pl.ds(base_t + j * c_sz, c_sz)], wv, sw)

        def proc(j, b, xg, wv, sg, sw):
            pltpu.make_async_copy(x_hbm.at[si_v[b]], xg, sg).wait()
            pltpu.make_async_copy(
                w_hbm.at[pl.ds(base_t + j * c_sz, c_sz)], wv, sw).wait()

            @plsc.parallel_loop(0, c_sz, unroll=4)
            def _mul(r):
                for kk in range(_F // _L):
                    sl = pl.ds(kk * _L, _L)
                    xg[r, sl] = xg[r, sl] * wv[r, sl]

            pltpu.sync_copy(xg, agg_sh.at[di_v[b]], add=True)

        gwbuf = [(xg_a, wv_a, sg_a, sw_a), (xg_b, wv_b, sg_b, sw_b)]

        for j in range(4):
            fire_idx(j, j)
        wait_idx(0, 0)
        fire_gw(0, 0, *gwbuf[0])

        def quad(q, carry):
            j0 = q * 4
            for u in range(4):
                j = j0 + u
                nxt = j + 1

                @pl.when(nxt < npc)
                def _gw_next():
                    wait_idx(nxt, (u + 1) % 4)
                    fire_gw(nxt, (u + 1) % 4, *gwbuf[(u + 1) % 2])

                proc(j, u, *gwbuf[u % 2])

                @pl.when(j + 4 < npc)
                def _refill():
                    fire_idx(j + 4, u)

            return carry

        lax.fori_loop(0, (npc - 1) // 4, quad, 0)
        proc(npc - 1, 0, *gwbuf[0])
        plsc.subcore_barrier()
        pltpu.sync_copy(
            agg_sh.at[pl.ds(sid * rows_pt, rows_pt)],
            out_hbm.at[cid, pl.ds(sid * rows_pt, rows_pt)])

        @pl.when(sid == 0)
        def _write_tail():
            pltpu.sync_copy(
                agg_sh.at[pl.ds(_NS * rows_pt, rows_rem)],
                out_hbm.at[cid, pl.ds(_NS * rows_pt, rows_rem)])

    return k(x, w, src, dst, zeros_blk)


def _comb_tc(x, agg2, Wsc_i, Wlin_i, wang_row):
    n = x.shape[0]
    blk = 400
    nb = n // blk

    def body(x_ref, agg_ref, wsc_ref, wlin_ref, wang_ref, o_ref):
        xv = x_ref[...]
        agg = (agg_ref[0] + agg_ref[1]) * _INV_SQRT_NN
        sc = jnp.dot(xv, wsc_ref[...], preferred_element_type=_f32)
        conv = jnp.dot(agg, wlin_ref[...], preferred_element_type=_f32)
        ang = 0.1 * jnp.sum(agg * wang_ref[...], axis=1, keepdims=True)
        o_ref[...] = jnp.cos(ang) * sc + jnp.sin(ang) * conv

    return pl.pallas_call(
        body,
        grid=(nb,),
        in_specs=[
            pl.BlockSpec((blk, _F), lambda j: (j, 0)),
            pl.BlockSpec((_NC, blk, _F), lambda j: (0, j, 0)),
            pl.BlockSpec((_F, _F), lambda j: (0, 0)),
            pl.BlockSpec((_F, _F), lambda j: (0, 0)),
            pl.BlockSpec((1, _F), lambda j: (0, 0)),
        ],
        out_specs=pl.BlockSpec((blk, _F), lambda j: (j, 0)),
        out_shape=jax.ShapeDtypeStruct((n, _F), _f32),
    )(x, agg2, Wsc_i, Wlin_i, wang_row)


def kernel(species, positions, edges, potential, efield,
           Wspecies, Wv, We, W1, b1, W2, Wsc, Wlin, Wang):
    n = species.shape[0]
    e = edges.shape[0]
    layers = W1.shape[0]

    src = edges[:, 0]
    dst = edges[:, 1]
    pos_flat = jnp.pad(positions, ((0, 0), (0, 1))).reshape(-1)

    r2 = _r2_sc(pos_flat, src, dst)

    vals_np = np.linspace(0.0, _CUTOFF, _NUM_BASIS, dtype=np.float32)
    step = float(vals_np[1] - vals_np[0])
    inv_step = 1.0 / step
    vals = jnp.asarray(vals_np).reshape(1, _NUM_BASIS)
    rs = jnp.linspace(0.0, _CUTOFF, 4001)[1:]
    dsamp = (rs[:, None] - vals[0][None, :]) * inv_step
    bs = jnp.exp(-dsamp * dsamp) / 1.12
    mean = jnp.mean(bs, axis=0)
    std = jnp.std(bs, axis=0, ddof=1)
    ascale = ((1.0 / 1.12) / std).reshape(1, _NUM_BASIS)
    cshift = (mean / std).reshape(1, _NUM_BASIS)

    w_all = _w_tc(r2, W1, b1, W2, vals, ascale, cshift, inv_step)
    x = _emb_tc(species, potential, efield, Wspecies, Wv, We)

    zeros_blk = jnp.zeros((n // _NS, _F), _f32)

    for i in range(layers):
        agg2 = _gms_sc(x, w_all[i], src, dst, zeros_blk, 80)
        x = _comb_tc(x, agg2, Wsc[i], Wlin[i], Wang[i].reshape(1, _F))
    return x
